# Initial kernel scaffold; baseline (speedup 1.0000x reference)
import jax
import jax.numpy as jnp
from jax.experimental import pallas as pl


def kernel(x, feature1, feature2, edge_index, batch, f1_w1, f1_w2, f2_w1, f2_w2, conv1_ll_w, conv1_ll_b, conv1_lr_w, conv2_ll_w, conv2_ll_b, conv2_lr_w, lin1_w, lin1_b, lin2_w, lin2_b, lin_cat_w, lin_cat_b, norm_w, norm_b, norm_ms, lins_w, lins_b, final_w, final_b):
    raise NotImplementedError("write your pallas kernel here")


# speedup vs baseline: 2.5565x; 2.5565x over previous
import functools

import jax
import jax.numpy as jnp
from jax import lax
from jax.experimental import pallas as pl
from jax.experimental.pallas import tpu as pltpu
from jax.experimental.pallas import tpu_sc as plsc

N_NODES = 10000
N_EDGES = 320000
HID = 128
N_GRAPHS = 1000

NC = 2
NS = 16
NW = NC * NS


_GCH = 80
_EPW = N_EDGES // NW
_GITERS = _EPW // _GCH


def _sc_gather(x, src):
    mesh = plsc.VectorSubcoreMesh(core_axis_name="c", subcore_axis_name="s")

    @functools.partial(
        pl.kernel,
        out_type=jax.ShapeDtypeStruct((N_EDGES, HID), jnp.float32),
        mesh=mesh,
        scratch_types=[
            pltpu.VMEM((_GCH,), jnp.int32),
            pltpu.VMEM((_GCH, HID), jnp.float32),
            pltpu.SemaphoreType.DMA,
        ],
    )
    def gather_kernel(x_hbm, src_hbm, g_hbm, idx_v, rows_v, sem):
        wid = lax.axis_index("s") * NC + lax.axis_index("c")
        base = wid * _EPW

        def body(j, carry):
            off = base + j * _GCH
            pltpu.sync_copy(src_hbm.at[pl.ds(off, _GCH)], idx_v)
            pltpu.async_copy(x_hbm.at[idx_v], rows_v, sem).wait()
            pltpu.sync_copy(rows_v, g_hbm.at[pl.ds(off, _GCH)])
            return carry

        lax.fori_loop(0, _GITERS, body, 0)

    return gather_kernel(x, src)


_SCH = 80
_EPT = N_EDGES // NS
_SITERS = _EPT // _SCH
_NCHK = N_NODES // _SCH
_CPT = -(-_NCHK // NS)


def _sc_scatter(msg1, msg2, dst):
    mesh = plsc.VectorSubcoreMesh(core_axis_name="c", subcore_axis_name="s")
    out_sds = jax.ShapeDtypeStruct((N_NODES, HID), jnp.float32)

    @functools.partial(
        pl.kernel,
        out_type=(out_sds, out_sds),
        mesh=mesh,
        scratch_types=[
            pltpu.VMEM((_SCH,), jnp.int32),
            pltpu.VMEM((_SCH, HID), jnp.float32),
            pltpu.VMEM_SHARED((N_NODES, HID), jnp.float32),
            pltpu.SemaphoreType.DMA,
        ],
    )
    def scatter_kernel(m1_hbm, m2_hbm, dst_hbm, a1_hbm, a2_hbm,
                       idx_v, m_v, agg_sh, sem):
        cid = lax.axis_index("c")
        tid = lax.axis_index("s")

        def run(m_hbm, a_hbm):
            def zrow(i, carry):
                def zcol(j, carry2):
                    m_v[i, pl.ds(j * 16, 16)] = jnp.zeros((16,), jnp.float32)
                    return carry2
                return lax.fori_loop(0, HID // 16, zcol, carry)
            lax.fori_loop(0, _SCH, zrow, 0)

            def zcp(k, carry):
                ck = tid + NS * k
                @pl.when(ck < _NCHK)
                def _():
                    pltpu.sync_copy(m_v, agg_sh.at[pl.ds(ck * _SCH, _SCH)])
                return carry
            lax.fori_loop(0, _CPT, zcp, 0)
            plsc.subcore_barrier()

            base = tid * _EPT
            def body(j, carry):
                off = base + j * _SCH
                pltpu.sync_copy(dst_hbm.at[pl.ds(off, _SCH)], idx_v)
                pltpu.sync_copy(m_hbm.at[pl.ds(off, _SCH)], m_v)
                pltpu.sync_copy(m_v, agg_sh.at[idx_v], add=True)
                return carry
            lax.fori_loop(0, _SITERS, body, 0)
            plsc.subcore_barrier()

            def wb(k, carry):
                ck = tid + NS * k
                @pl.when(ck < _NCHK)
                def _():
                    rr = ck * _SCH
                    pltpu.sync_copy(agg_sh.at[pl.ds(rr, _SCH)], m_v)
                    pltpu.sync_copy(m_v, a_hbm.at[pl.ds(rr, _SCH)])
                return carry
            lax.fori_loop(0, _CPT, wb, 0)

        @pl.when(cid == 0)
        def _():
            run(m1_hbm, a1_hbm)

        @pl.when(cid == 1)
        def _():
            run(m2_hbm, a2_hbm)

    return scatter_kernel(msg1, msg2, dst)


_EB = 3200


def _tc_msg(feature1, feature2, g, w11t, w12t, w21t, w22t):
    grid = (N_EDGES // _EB,)

    def body(f1_ref, f2_ref, g_ref, w11_ref, w12_ref, w21_ref, w22_ref,
             m1_ref, m2_ref):
        gg = g_ref[...]
        u1 = jnp.dot(f1_ref[...], w11_ref[...],
                     preferred_element_type=jnp.float32)
        e1 = jnp.dot(u1, w12_ref[...], preferred_element_type=jnp.float32)
        m1_ref[...] = e1 * gg
        u2 = jnp.dot(f2_ref[...], w21_ref[...],
                     preferred_element_type=jnp.float32)
        e2 = jnp.dot(u2, w22_ref[...], preferred_element_type=jnp.float32)
        m2_ref[...] = e2 * gg

    eb_spec = lambda w: pl.BlockSpec((_EB, w), lambda i: (i, 0))
    full = lambda a, b: pl.BlockSpec((a, b), lambda i: (0, 0))
    out_sds = jax.ShapeDtypeStruct((N_EDGES, HID), jnp.float32)
    return pl.pallas_call(
        body,
        grid=grid,
        in_specs=[
            eb_spec(feature1.shape[1]),
            eb_spec(feature2.shape[1]),
            eb_spec(HID),
            full(*w11t.shape), full(*w12t.shape),
            full(*w21t.shape), full(*w22t.shape),
        ],
        out_specs=[eb_spec(HID), eb_spec(HID)],
        out_shape=[out_sds, out_sds],
    )(feature1, feature2, g, w11t, w12t, w21t, w22t)


_NB = 2000


def _swish(v):
    return v / (1.0 + jnp.exp(-v))


def _tc_node1(x, agg1, agg2, batf, c1llt, c1ll_b, c1lrt, c2llt, c2ll_b, c2lrt,
              lin1t, lin1_b, lin2t, lin2_b, lcatt, lcat_b,
              l0t, l0b, l1t, l1b, l2t, l2b):
    grid = (N_NODES // _NB,)

    def body(x_ref, a1_ref, a2_ref, b_ref,
             c1llt_r, c1llb_r, c1lrt_r, c2llt_r, c2llb_r, c2lrt_r,
             lin1t_r, lin1b_r, lin2t_r, lin2b_r, lcatt_r, lcatb_r,
             l0t_r, l0b_r, l1t_r, l1b_r, l2t_r, l2b_r,
             h_ref, s0_ref, s1_ref, s2_ref):
        i = pl.program_id(0)
        xx = x_ref[...]
        h1 = (jnp.dot(a1_ref[...], c1llt_r[...],
                      preferred_element_type=jnp.float32) + c1llb_r[...]
              + jnp.dot(xx, c1lrt_r[...], preferred_element_type=jnp.float32))
        h1 = _swish(jnp.dot(h1, lin1t_r[...],
                            preferred_element_type=jnp.float32) + lin1b_r[...])
        h2 = (jnp.dot(a2_ref[...], c2llt_r[...],
                      preferred_element_type=jnp.float32) + c2llb_r[...]
              + jnp.dot(xx, c2lrt_r[...], preferred_element_type=jnp.float32))
        h2 = _swish(jnp.dot(h2, lin2t_r[...],
                            preferred_element_type=jnp.float32) + lin2b_r[...])
        hc = jnp.concatenate([h1, h2], axis=1)
        h = jnp.dot(hc, lcatt_r[...],
                    preferred_element_type=jnp.float32) + lcatb_r[...] + xx
        for wt_r, wb_r in ((l0t_r, l0b_r), (l1t_r, l1b_r), (l2t_r, l2b_r)):
            h = _swish(jnp.dot(h, wt_r[...],
                               preferred_element_type=jnp.float32)
                       + wb_r[...]) + h
        h_ref[...] = h

        gids = lax.broadcasted_iota(jnp.int32, (_NB, N_GRAPHS), 1
                                    ).astype(jnp.float32)
        oh = (b_ref[...] == gids).astype(jnp.float32)
        ones = jnp.ones((_NB, HID), jnp.float32)
        dimn = (((0,), (0,)), ((), ()))
        ps0 = lax.dot_general(oh, ones, dimn,
                              preferred_element_type=jnp.float32)
        ps1 = lax.dot_general(oh, h, dimn, preferred_element_type=jnp.float32)
        ps2 = lax.dot_general(oh, h * h, dimn,
                              preferred_element_type=jnp.float32)

        @pl.when(i == 0)
        def _():
            s0_ref[...] = jnp.zeros_like(s0_ref)
            s1_ref[...] = jnp.zeros_like(s1_ref)
            s2_ref[...] = jnp.zeros_like(s2_ref)

        s0_ref[...] += ps0
        s1_ref[...] += ps1
        s2_ref[...] += ps2

    nb = lambda w: pl.BlockSpec((_NB, w), lambda i: (i, 0))
    full = lambda a, b: pl.BlockSpec((a, b), lambda i: (0, 0))
    stat = pl.BlockSpec((N_GRAPHS, HID), lambda i: (0, 0))
    sds = jax.ShapeDtypeStruct
    return pl.pallas_call(
        body,
        grid=grid,
        in_specs=[
            nb(HID), nb(HID), nb(HID), nb(1),
            full(HID, HID), full(1, HID), full(HID, HID),
            full(HID, HID), full(1, HID), full(HID, HID),
            full(HID, HID), full(1, HID), full(HID, HID), full(1, HID),
            full(2 * HID, HID), full(1, HID),
            full(HID, HID), full(1, HID), full(HID, HID), full(1, HID),
            full(HID, HID), full(1, HID),
        ],
        out_specs=[nb(HID), stat, stat, stat],
        out_shape=[
            sds((N_NODES, HID), jnp.float32),
            sds((N_GRAPHS, HID), jnp.float32),
            sds((N_GRAPHS, HID), jnp.float32),
            sds((N_GRAPHS, HID), jnp.float32),
        ],
    )(x, agg1, agg2, batf, c1llt, c1ll_b, c1lrt, c2llt, c2ll_b, c2lrt,
      lin1t, lin1_b, lin2t, lin2_b, lcatt, lcat_b, l0t, l0b, l1t, l1b,
      l2t, l2b)


def _tc_node2(h, batf, s0, s1, s2, norm_w, norm_b, norm_ms, finalt, final_b):
    grid = (N_NODES // _NB,)

    def body(h_ref, b_ref, s0_ref, s1_ref, s2_ref,
             nw_r, nb_r, nms_r, ft_r, fb_r, o_ref):
        cnt = jnp.maximum(s0_ref[...], 1.0)
        mean = s1_ref[...] / cnt
        eh2 = s2_ref[...] / cnt
        ms = nms_r[...]
        var = eh2 - (2.0 * ms - ms * ms) * (mean * mean)
        gids = lax.broadcasted_iota(jnp.int32, (_NB, N_GRAPHS), 1
                                    ).astype(jnp.float32)
        oh = (b_ref[...] == gids).astype(jnp.float32)
        meanb = jnp.dot(oh, mean, preferred_element_type=jnp.float32)
        varb = jnp.dot(oh, var, preferred_element_type=jnp.float32)
        h = h_ref[...]
        centered = h - meanb * ms
        normed = (nw_r[...] * centered * lax.rsqrt(varb + 1e-5) + nb_r[...])
        o_ref[...] = jnp.dot(normed, ft_r[...],
                             preferred_element_type=jnp.float32) + fb_r[...]

    nb = lambda w: pl.BlockSpec((_NB, w), lambda i: (i, 0))
    full = lambda a, b: pl.BlockSpec((a, b), lambda i: (0, 0))
    stat = pl.BlockSpec((N_GRAPHS, HID), lambda i: (0, 0))
    return pl.pallas_call(
        body,
        grid=grid,
        in_specs=[
            nb(HID), nb(1), stat, stat, stat,
            full(1, HID), full(1, HID), full(1, HID),
            full(HID, HID), full(1, HID),
        ],
        out_specs=nb(HID),
        out_shape=jax.ShapeDtypeStruct((N_NODES, HID), jnp.float32),
    )(h, batf, s0, s1, s2, norm_w, norm_b, norm_ms, finalt, final_b)


def kernel(x, feature1, feature2, edge_index, batch,
           f1_w1, f1_w2, f2_w1, f2_w2,
           conv1_ll_w, conv1_ll_b, conv1_lr_w,
           conv2_ll_w, conv2_ll_b, conv2_lr_w,
           lin1_w, lin1_b, lin2_w, lin2_b,
           lin_cat_w, lin_cat_b,
           norm_w, norm_b, norm_ms,
           lins_w, lins_b, final_w, final_b):
    src = edge_index[0].astype(jnp.int32)
    dst = edge_index[1].astype(jnp.int32)
    batf = batch.astype(jnp.float32).reshape(N_NODES, 1)

    g = _sc_gather(x, src)

    msg1, msg2 = _tc_msg(feature1, feature2, g,
                         f1_w1.T, f1_w2.T, f2_w1.T, f2_w2.T)

    agg1, agg2 = _sc_scatter(msg1, msg2, dst)

    r2 = lambda b: b.reshape(1, HID)
    h, s0, s1, s2 = _tc_node1(
        x, agg1, agg2, batf,
        conv1_ll_w.T, r2(conv1_ll_b), conv1_lr_w.T,
        conv2_ll_w.T, r2(conv2_ll_b), conv2_lr_w.T,
        lin1_w.T, r2(lin1_b), lin2_w.T, r2(lin2_b),
        lin_cat_w.T, r2(lin_cat_b),
        lins_w[0].T, lins_b[0].reshape(1, HID),
        lins_w[1].T, lins_b[1].reshape(1, HID),
        lins_w[2].T, lins_b[2].reshape(1, HID),
    )
    out = _tc_node2(h, batf, s0, s1, s2,
                    r2(norm_w), r2(norm_b), r2(norm_ms),
                    final_w.T, final_b.reshape(1, HID))
    return out


# speedup vs baseline: 3.3516x; 1.3110x over previous
import functools

import jax
import jax.numpy as jnp
from jax import lax
from jax.experimental import pallas as pl
from jax.experimental.pallas import tpu as pltpu
from jax.experimental.pallas import tpu_sc as plsc

N_NODES = 10000
N_EDGES = 320000
HID = 128
N_GRAPHS = 1000

NC = 2
NS = 16
NW = NC * NS


_GCH = 80
_EPW = N_EDGES // NW
_GITERS = _EPW // _GCH


def _sc_gather(x, src):
    mesh = plsc.VectorSubcoreMesh(core_axis_name="c", subcore_axis_name="s")

    @functools.partial(
        pl.kernel,
        out_type=jax.ShapeDtypeStruct((N_EDGES, HID), jnp.float32),
        mesh=mesh,
        scratch_types=[
            pltpu.VMEM((_GCH,), jnp.int32),
            pltpu.VMEM((_GCH,), jnp.int32),
            pltpu.VMEM((_GCH, HID), jnp.float32),
            pltpu.VMEM((_GCH, HID), jnp.float32),
            pltpu.SemaphoreType.DMA,
            pltpu.SemaphoreType.DMA,
            pltpu.SemaphoreType.DMA,
        ],
    )
    def gather_kernel(x_hbm, src_hbm, g_hbm, idx0, idx1, rows0, rows1,
                      semi, semg, semw):
        wid = lax.axis_index("s") * NC + lax.axis_index("c")
        base = wid * _EPW
        idxs = (idx0, idx1)
        rows = (rows0, rows1)

        def chunk(c, b, has_prev2, has_next):
            off = base + c * _GCH
            if has_prev2:
                pltpu.make_async_copy(
                    rows[b], g_hbm.at[pl.ds(off - 2 * _GCH, _GCH)],
                    semw).wait()
            pltpu.make_async_copy(
                src_hbm.at[pl.ds(off, _GCH)], idxs[b], semi).wait()
            if has_next:
                pltpu.async_copy(
                    src_hbm.at[pl.ds(off + _GCH, _GCH)], idxs[b ^ 1], semi)
            pltpu.async_copy(x_hbm.at[idxs[b]], rows[b], semg).wait()
            pltpu.async_copy(rows[b], g_hbm.at[pl.ds(off, _GCH)], semw)

        pltpu.async_copy(src_hbm.at[pl.ds(base, _GCH)], idx0, semi)
        chunk(0, 0, False, True)
        chunk(1, 1, False, True)

        def body(k, carry):
            c = 2 * k
            chunk(c, 0, True, True)
            chunk(c + 1, 1, True, True)
            return carry

        lax.fori_loop(1, (_GITERS - 1) // 2, body, 0)
        chunk(_GITERS - 1, 0, True, False)
        pltpu.make_async_copy(
            rows1, g_hbm.at[pl.ds(base + (_GITERS - 2) * _GCH, _GCH)],
            semw).wait()
        pltpu.make_async_copy(
            rows0, g_hbm.at[pl.ds(base + (_GITERS - 1) * _GCH, _GCH)],
            semw).wait()

    return gather_kernel(x, src)


_SCH = 80
_EPT = N_EDGES // NS
_SITERS = _EPT // _SCH
_NCHK = N_NODES // _SCH
_CPT = -(-_NCHK // NS)


def _sc_scatter(msg1, msg2, dst):
    mesh = plsc.VectorSubcoreMesh(core_axis_name="c", subcore_axis_name="s")
    out_sds = jax.ShapeDtypeStruct((N_NODES, HID), jnp.float32)

    @functools.partial(
        pl.kernel,
        out_type=(out_sds, out_sds),
        mesh=mesh,
        scratch_types=[
            pltpu.VMEM((_SCH,), jnp.int32),
            pltpu.VMEM((_SCH,), jnp.int32),
            pltpu.VMEM((_SCH, HID), jnp.float32),
            pltpu.VMEM((_SCH, HID), jnp.float32),
            pltpu.VMEM_SHARED((N_NODES, HID), jnp.float32),
            pltpu.SemaphoreType.DMA,
            pltpu.SemaphoreType.DMA,
        ],
    )
    def scatter_kernel(m1_hbm, m2_hbm, dst_hbm, a1_hbm, a2_hbm,
                       idx0, idx1, m0, m1, agg_sh, semi, semm):
        cid = lax.axis_index("c")
        tid = lax.axis_index("s")
        idxs = (idx0, idx1)
        ms = (m0, m1)

        def run(m_hbm, a_hbm):
            def zrow(i, carry):
                def zcol(j, carry2):
                    m0[i, pl.ds(j * 16, 16)] = jnp.zeros((16,), jnp.float32)
                    return carry2
                return lax.fori_loop(0, HID // 16, zcol, carry)
            lax.fori_loop(0, _SCH, zrow, 0)

            def zcp(k, carry):
                ck = tid + NS * k
                @pl.when(ck < _NCHK)
                def _():
                    pltpu.sync_copy(m0, agg_sh.at[pl.ds(ck * _SCH, _SCH)])
                return carry
            lax.fori_loop(0, _CPT, zcp, 0)
            plsc.subcore_barrier()

            base = tid * _EPT

            def chunk(c, b, has_next):
                off = base + c * _SCH
                pltpu.make_async_copy(
                    dst_hbm.at[pl.ds(off, _SCH)], idxs[b], semi).wait()
                pltpu.make_async_copy(
                    m_hbm.at[pl.ds(off, _SCH)], ms[b], semm).wait()
                if has_next:
                    pltpu.async_copy(
                        dst_hbm.at[pl.ds(off + _SCH, _SCH)], idxs[b ^ 1],
                        semi)
                    pltpu.async_copy(
                        m_hbm.at[pl.ds(off + _SCH, _SCH)], ms[b ^ 1], semm)
                pltpu.sync_copy(ms[b], agg_sh.at[idxs[b]], add=True)

            pltpu.async_copy(dst_hbm.at[pl.ds(base, _SCH)], idx0, semi)
            pltpu.async_copy(m_hbm.at[pl.ds(base, _SCH)], m0, semm)

            def body(k, carry):
                c = 2 * k
                chunk(c, 0, True)
                chunk(c + 1, 1, True)
                return carry

            lax.fori_loop(0, _SITERS // 2 - 1, body, 0)
            chunk(_SITERS - 2, 0, True)
            chunk(_SITERS - 1, 1, False)
            plsc.subcore_barrier()

            def wb(k, carry):
                ck = tid + NS * k
                @pl.when(ck < _NCHK)
                def _():
                    rr = ck * _SCH
                    pltpu.sync_copy(agg_sh.at[pl.ds(rr, _SCH)], m0)
                    pltpu.sync_copy(m0, a_hbm.at[pl.ds(rr, _SCH)])
                return carry
            lax.fori_loop(0, _CPT, wb, 0)

        @pl.when(cid == 0)
        def _():
            run(m1_hbm, a1_hbm)

        @pl.when(cid == 1)
        def _():
            run(m2_hbm, a2_hbm)

    return scatter_kernel(msg1, msg2, dst)


_EB = 3200


def _tc_msg(feature1, feature2, g, w11t, w12t, w21t, w22t):
    grid = (N_EDGES // _EB,)

    def body(f1_ref, f2_ref, g_ref, w11_ref, w12_ref, w21_ref, w22_ref,
             m1_ref, m2_ref):
        gg = g_ref[...]
        u1 = jnp.dot(f1_ref[...], w11_ref[...],
                     preferred_element_type=jnp.float32)
        e1 = jnp.dot(u1, w12_ref[...], preferred_element_type=jnp.float32)
        m1_ref[...] = e1 * gg
        u2 = jnp.dot(f2_ref[...], w21_ref[...],
                     preferred_element_type=jnp.float32)
        e2 = jnp.dot(u2, w22_ref[...], preferred_element_type=jnp.float32)
        m2_ref[...] = e2 * gg

    eb_spec = lambda w: pl.BlockSpec((_EB, w), lambda i: (i, 0))
    full = lambda a, b: pl.BlockSpec((a, b), lambda i: (0, 0))
    out_sds = jax.ShapeDtypeStruct((N_EDGES, HID), jnp.float32)
    return pl.pallas_call(
        body,
        grid=grid,
        in_specs=[
            eb_spec(feature1.shape[1]),
            eb_spec(feature2.shape[1]),
            eb_spec(HID),
            full(*w11t.shape), full(*w12t.shape),
            full(*w21t.shape), full(*w22t.shape),
        ],
        out_specs=[eb_spec(HID), eb_spec(HID)],
        out_shape=[out_sds, out_sds],
    )(feature1, feature2, g, w11t, w12t, w21t, w22t)


_NB = 2000


def _swish(v):
    return v / (1.0 + jnp.exp(-v))


def _tc_node1(x, agg1, agg2, batf, c1llt, c1ll_b, c1lrt, c2llt, c2ll_b, c2lrt,
              lin1t, lin1_b, lin2t, lin2_b, lcatt, lcat_b,
              l0t, l0b, l1t, l1b, l2t, l2b):
    grid = (N_NODES // _NB,)

    def body(x_ref, a1_ref, a2_ref, b_ref,
             c1llt_r, c1llb_r, c1lrt_r, c2llt_r, c2llb_r, c2lrt_r,
             lin1t_r, lin1b_r, lin2t_r, lin2b_r, lcatt_r, lcatb_r,
             l0t_r, l0b_r, l1t_r, l1b_r, l2t_r, l2b_r,
             h_ref, s0_ref, s1_ref, s2_ref):
        i = pl.program_id(0)
        xx = x_ref[...]
        h1 = (jnp.dot(a1_ref[...], c1llt_r[...],
                      preferred_element_type=jnp.float32) + c1llb_r[...]
              + jnp.dot(xx, c1lrt_r[...], preferred_element_type=jnp.float32))
        h1 = _swish(jnp.dot(h1, lin1t_r[...],
                            preferred_element_type=jnp.float32) + lin1b_r[...])
        h2 = (jnp.dot(a2_ref[...], c2llt_r[...],
                      preferred_element_type=jnp.float32) + c2llb_r[...]
              + jnp.dot(xx, c2lrt_r[...], preferred_element_type=jnp.float32))
        h2 = _swish(jnp.dot(h2, lin2t_r[...],
                            preferred_element_type=jnp.float32) + lin2b_r[...])
        hc = jnp.concatenate([h1, h2], axis=1)
        h = jnp.dot(hc, lcatt_r[...],
                    preferred_element_type=jnp.float32) + lcatb_r[...] + xx
        for wt_r, wb_r in ((l0t_r, l0b_r), (l1t_r, l1b_r), (l2t_r, l2b_r)):
            h = _swish(jnp.dot(h, wt_r[...],
                               preferred_element_type=jnp.float32)
                       + wb_r[...]) + h
        h_ref[...] = h

        gids = lax.broadcasted_iota(jnp.int32, (_NB, N_GRAPHS), 1
                                    ).astype(jnp.float32)
        oh = (b_ref[...] == gids).astype(jnp.float32)
        ones = jnp.ones((_NB, HID), jnp.float32)
        dimn = (((0,), (0,)), ((), ()))
        ps0 = lax.dot_general(oh, ones, dimn,
                              preferred_element_type=jnp.float32)
        ps1 = lax.dot_general(oh, h, dimn, preferred_element_type=jnp.float32)
        ps2 = lax.dot_general(oh, h * h, dimn,
                              preferred_element_type=jnp.float32)

        @pl.when(i == 0)
        def _():
            s0_ref[...] = jnp.zeros_like(s0_ref)
            s1_ref[...] = jnp.zeros_like(s1_ref)
            s2_ref[...] = jnp.zeros_like(s2_ref)

        s0_ref[...] += ps0
        s1_ref[...] += ps1
        s2_ref[...] += ps2

    nb = lambda w: pl.BlockSpec((_NB, w), lambda i: (i, 0))
    full = lambda a, b: pl.BlockSpec((a, b), lambda i: (0, 0))
    stat = pl.BlockSpec((N_GRAPHS, HID), lambda i: (0, 0))
    sds = jax.ShapeDtypeStruct
    return pl.pallas_call(
        body,
        grid=grid,
        in_specs=[
            nb(HID), nb(HID), nb(HID), nb(1),
            full(HID, HID), full(1, HID), full(HID, HID),
            full(HID, HID), full(1, HID), full(HID, HID),
            full(HID, HID), full(1, HID), full(HID, HID), full(1, HID),
            full(2 * HID, HID), full(1, HID),
            full(HID, HID), full(1, HID), full(HID, HID), full(1, HID),
            full(HID, HID), full(1, HID),
        ],
        out_specs=[nb(HID), stat, stat, stat],
        out_shape=[
            sds((N_NODES, HID), jnp.float32),
            sds((N_GRAPHS, HID), jnp.float32),
            sds((N_GRAPHS, HID), jnp.float32),
            sds((N_GRAPHS, HID), jnp.float32),
        ],
    )(x, agg1, agg2, batf, c1llt, c1ll_b, c1lrt, c2llt, c2ll_b, c2lrt,
      lin1t, lin1_b, lin2t, lin2_b, lcatt, lcat_b, l0t, l0b, l1t, l1b,
      l2t, l2b)


def _tc_node2(h, batf, s0, s1, s2, norm_w, norm_b, norm_ms, finalt, final_b):
    grid = (N_NODES // _NB,)

    def body(h_ref, b_ref, s0_ref, s1_ref, s2_ref,
             nw_r, nb_r, nms_r, ft_r, fb_r, o_ref):
        cnt = jnp.maximum(s0_ref[...], 1.0)
        mean = s1_ref[...] / cnt
        eh2 = s2_ref[...] / cnt
        ms = nms_r[...]
        var = eh2 - (2.0 * ms - ms * ms) * (mean * mean)
        gids = lax.broadcasted_iota(jnp.int32, (_NB, N_GRAPHS), 1
                                    ).astype(jnp.float32)
        oh = (b_ref[...] == gids).astype(jnp.float32)
        meanb = jnp.dot(oh, mean, preferred_element_type=jnp.float32)
        varb = jnp.dot(oh, var, preferred_element_type=jnp.float32)
        h = h_ref[...]
        centered = h - meanb * ms
        normed = (nw_r[...] * centered * lax.rsqrt(varb + 1e-5) + nb_r[...])
        o_ref[...] = jnp.dot(normed, ft_r[...],
                             preferred_element_type=jnp.float32) + fb_r[...]

    nb = lambda w: pl.BlockSpec((_NB, w), lambda i: (i, 0))
    full = lambda a, b: pl.BlockSpec((a, b), lambda i: (0, 0))
    stat = pl.BlockSpec((N_GRAPHS, HID), lambda i: (0, 0))
    return pl.pallas_call(
        body,
        grid=grid,
        in_specs=[
            nb(HID), nb(1), stat, stat, stat,
            full(1, HID), full(1, HID), full(1, HID),
            full(HID, HID), full(1, HID),
        ],
        out_specs=nb(HID),
        out_shape=jax.ShapeDtypeStruct((N_NODES, HID), jnp.float32),
    )(h, batf, s0, s1, s2, norm_w, norm_b, norm_ms, finalt, final_b)


def kernel(x, feature1, feature2, edge_index, batch,
           f1_w1, f1_w2, f2_w1, f2_w2,
           conv1_ll_w, conv1_ll_b, conv1_lr_w,
           conv2_ll_w, conv2_ll_b, conv2_lr_w,
           lin1_w, lin1_b, lin2_w, lin2_b,
           lin_cat_w, lin_cat_b,
           norm_w, norm_b, norm_ms,
           lins_w, lins_b, final_w, final_b):
    src = edge_index[0].astype(jnp.int32)
    dst = edge_index[1].astype(jnp.int32)
    batf = batch.astype(jnp.float32).reshape(N_NODES, 1)

    g = _sc_gather(x, src)

    msg1, msg2 = _tc_msg(feature1, feature2, g,
                         f1_w1.T, f1_w2.T, f2_w1.T, f2_w2.T)

    agg1, agg2 = _sc_scatter(msg1, msg2, dst)

    r2 = lambda b: b.reshape(1, HID)
    h, s0, s1, s2 = _tc_node1(
        x, agg1, agg2, batf,
        conv1_ll_w.T, r2(conv1_ll_b), conv1_lr_w.T,
        conv2_ll_w.T, r2(conv2_ll_b), conv2_lr_w.T,
        lin1_w.T, r2(lin1_b), lin2_w.T, r2(lin2_b),
        lin_cat_w.T, r2(lin_cat_b),
        lins_w[0].T, lins_b[0].reshape(1, HID),
        lins_w[1].T, lins_b[1].reshape(1, HID),
        lins_w[2].T, lins_b[2].reshape(1, HID),
    )
    out = _tc_node2(h, batf, s0, s1, s2,
                    r2(norm_w), r2(norm_b), r2(norm_ms),
                    final_w.T, final_b.reshape(1, HID))
    return out


# speedup vs baseline: 3.3753x; 1.0071x over previous
import functools

import jax
import jax.numpy as jnp
from jax import lax
from jax.experimental import pallas as pl
from jax.experimental.pallas import tpu as pltpu
from jax.experimental.pallas import tpu_sc as plsc

N_NODES = 10000
N_EDGES = 320000
HID = 128
N_GRAPHS = 1000

NC = 2
NS = 16
NW = NC * NS


_GCH = 80
_EPW = N_EDGES // NW
_GITERS = _EPW // _GCH


def _sc_gather(x, src):
    mesh = plsc.VectorSubcoreMesh(core_axis_name="c", subcore_axis_name="s")

    @functools.partial(
        pl.kernel,
        out_type=jax.ShapeDtypeStruct((N_EDGES, HID), jnp.float32),
        mesh=mesh,
        scratch_types=[
            pltpu.VMEM((_GCH,), jnp.int32),
            pltpu.VMEM((_GCH,), jnp.int32),
            pltpu.VMEM((_GCH, HID), jnp.float32),
            pltpu.VMEM((_GCH, HID), jnp.float32),
            pltpu.SemaphoreType.DMA,
            pltpu.SemaphoreType.DMA,
            pltpu.SemaphoreType.DMA,
        ],
    )
    def gather_kernel(x_hbm, src_hbm, g_hbm, idx0, idx1, rows0, rows1,
                      semi, semg, semw):
        wid = lax.axis_index("s") * NC + lax.axis_index("c")
        base = wid * _EPW
        idxs = (idx0, idx1)
        rows = (rows0, rows1)

        def chunk(c, b, has_prev2, has_next):
            off = base + c * _GCH
            if has_prev2:
                pltpu.make_async_copy(
                    rows[b], g_hbm.at[pl.ds(off - 2 * _GCH, _GCH)],
                    semw).wait()
            pltpu.make_async_copy(
                src_hbm.at[pl.ds(off, _GCH)], idxs[b], semi).wait()
            if has_next:
                pltpu.async_copy(
                    src_hbm.at[pl.ds(off + _GCH, _GCH)], idxs[b ^ 1], semi)
            pltpu.async_copy(x_hbm.at[idxs[b]], rows[b], semg).wait()
            pltpu.async_copy(rows[b], g_hbm.at[pl.ds(off, _GCH)], semw)

        pltpu.async_copy(src_hbm.at[pl.ds(base, _GCH)], idx0, semi)
        chunk(0, 0, False, True)
        chunk(1, 1, False, True)

        def body(k, carry):
            c = 2 * k
            chunk(c, 0, True, True)
            chunk(c + 1, 1, True, True)
            return carry

        lax.fori_loop(1, (_GITERS - 1) // 2, body, 0)
        chunk(_GITERS - 1, 0, True, False)
        pltpu.make_async_copy(
            rows1, g_hbm.at[pl.ds(base + (_GITERS - 2) * _GCH, _GCH)],
            semw).wait()
        pltpu.make_async_copy(
            rows0, g_hbm.at[pl.ds(base + (_GITERS - 1) * _GCH, _GCH)],
            semw).wait()

    return gather_kernel(x, src)


_SCH = 80
_EPT = N_EDGES // NS
_SITERS = _EPT // _SCH
_NCHK = N_NODES // _SCH
_CPT = -(-_NCHK // NS)


def _sc_scatter(msg1, msg2, dst):
    mesh = plsc.VectorSubcoreMesh(core_axis_name="c", subcore_axis_name="s")
    out_sds = jax.ShapeDtypeStruct((N_NODES, HID), jnp.float32)

    @functools.partial(
        pl.kernel,
        out_type=(out_sds, out_sds),
        mesh=mesh,
        scratch_types=[
            pltpu.VMEM((_SCH,), jnp.int32),
            pltpu.VMEM((_SCH,), jnp.int32),
            pltpu.VMEM((_SCH, HID), jnp.float32),
            pltpu.VMEM((_SCH, HID), jnp.float32),
            pltpu.VMEM_SHARED((N_NODES, HID), jnp.float32),
            pltpu.SemaphoreType.DMA,
            pltpu.SemaphoreType.DMA,
        ],
    )
    def scatter_kernel(m1_hbm, m2_hbm, dst_hbm, a1_hbm, a2_hbm,
                       idx0, idx1, m0, m1, agg_sh, semi, semm):
        cid = lax.axis_index("c")
        tid = lax.axis_index("s")
        idxs = (idx0, idx1)
        ms = (m0, m1)

        def run(m_hbm, a_hbm):
            def zrow(i, carry):
                def zcol(j, carry2):
                    m0[i, pl.ds(j * 16, 16)] = jnp.zeros((16,), jnp.float32)
                    return carry2
                return lax.fori_loop(0, HID // 16, zcol, carry)
            lax.fori_loop(0, _SCH, zrow, 0)

            def zcp(k, carry):
                ck = tid + NS * k
                @pl.when(ck < _NCHK)
                def _():
                    pltpu.sync_copy(m0, agg_sh.at[pl.ds(ck * _SCH, _SCH)])
                return carry
            lax.fori_loop(0, _CPT, zcp, 0)
            plsc.subcore_barrier()

            base = tid * _EPT

            def chunk(c, b, has_next):
                off = base + c * _SCH
                pltpu.make_async_copy(
                    dst_hbm.at[pl.ds(off, _SCH)], idxs[b], semi).wait()
                pltpu.make_async_copy(
                    m_hbm.at[pl.ds(off, _SCH)], ms[b], semm).wait()
                if has_next:
                    pltpu.async_copy(
                        dst_hbm.at[pl.ds(off + _SCH, _SCH)], idxs[b ^ 1],
                        semi)
                    pltpu.async_copy(
                        m_hbm.at[pl.ds(off + _SCH, _SCH)], ms[b ^ 1], semm)
                pltpu.sync_copy(ms[b], agg_sh.at[idxs[b]], add=True)

            pltpu.async_copy(dst_hbm.at[pl.ds(base, _SCH)], idx0, semi)
            pltpu.async_copy(m_hbm.at[pl.ds(base, _SCH)], m0, semm)

            def body(k, carry):
                c = 2 * k
                chunk(c, 0, True)
                chunk(c + 1, 1, True)
                return carry

            lax.fori_loop(0, _SITERS // 2 - 1, body, 0)
            chunk(_SITERS - 2, 0, True)
            chunk(_SITERS - 1, 1, False)
            plsc.subcore_barrier()

            def wb(k, carry):
                ck = tid + NS * k
                @pl.when(ck < _NCHK)
                def _():
                    rr = ck * _SCH
                    pltpu.sync_copy(agg_sh.at[pl.ds(rr, _SCH)], m0)
                    pltpu.sync_copy(m0, a_hbm.at[pl.ds(rr, _SCH)])
                return carry
            lax.fori_loop(0, _CPT, wb, 0)

        @pl.when(cid == 0)
        def _():
            run(m1_hbm, a1_hbm)

        @pl.when(cid == 1)
        def _():
            run(m2_hbm, a2_hbm)

    return scatter_kernel(msg1, msg2, dst)


_EB = 6400


def _tc_msg(feature1, feature2, g, cw1t, cw2t):
    grid = (N_EDGES // _EB,)

    def body(f1_ref, f2_ref, g_ref, cw1_ref, cw2_ref, m1_ref, m2_ref):
        gg = g_ref[...]
        e1 = jnp.dot(f1_ref[...], cw1_ref[...],
                     preferred_element_type=jnp.float32)
        m1_ref[...] = e1 * gg
        e2 = jnp.dot(f2_ref[...], cw2_ref[...],
                     preferred_element_type=jnp.float32)
        m2_ref[...] = e2 * gg

    eb_spec = lambda w: pl.BlockSpec((_EB, w), lambda i: (i, 0))
    full = lambda a, b: pl.BlockSpec((a, b), lambda i: (0, 0))
    out_sds = jax.ShapeDtypeStruct((N_EDGES, HID), jnp.float32)
    return pl.pallas_call(
        body,
        grid=grid,
        in_specs=[
            eb_spec(feature1.shape[1]),
            eb_spec(feature2.shape[1]),
            eb_spec(HID),
            full(*cw1t.shape), full(*cw2t.shape),
        ],
        out_specs=[eb_spec(HID), eb_spec(HID)],
        out_shape=[out_sds, out_sds],
    )(feature1, feature2, g, cw1t, cw2t)


_NB = 2000


def _swish(v):
    return v / (1.0 + jnp.exp(-v))


def _tc_node1(x, agg1, agg2, batf, c1llt, c1ll_b, c1lrt, c2llt, c2ll_b, c2lrt,
              lin1t, lin1_b, lin2t, lin2_b, lcatt, lcat_b,
              l0t, l0b, l1t, l1b, l2t, l2b):
    grid = (N_NODES // _NB,)

    def body(x_ref, a1_ref, a2_ref, b_ref,
             c1llt_r, c1llb_r, c1lrt_r, c2llt_r, c2llb_r, c2lrt_r,
             lin1t_r, lin1b_r, lin2t_r, lin2b_r, lcatt_r, lcatb_r,
             l0t_r, l0b_r, l1t_r, l1b_r, l2t_r, l2b_r,
             h_ref, s0_ref, s1_ref, s2_ref):
        i = pl.program_id(0)
        xx = x_ref[...]
        h1 = (jnp.dot(a1_ref[...], c1llt_r[...],
                      preferred_element_type=jnp.float32) + c1llb_r[...]
              + jnp.dot(xx, c1lrt_r[...], preferred_element_type=jnp.float32))
        h1 = _swish(jnp.dot(h1, lin1t_r[...],
                            preferred_element_type=jnp.float32) + lin1b_r[...])
        h2 = (jnp.dot(a2_ref[...], c2llt_r[...],
                      preferred_element_type=jnp.float32) + c2llb_r[...]
              + jnp.dot(xx, c2lrt_r[...], preferred_element_type=jnp.float32))
        h2 = _swish(jnp.dot(h2, lin2t_r[...],
                            preferred_element_type=jnp.float32) + lin2b_r[...])
        hc = jnp.concatenate([h1, h2], axis=1)
        h = jnp.dot(hc, lcatt_r[...],
                    preferred_element_type=jnp.float32) + lcatb_r[...] + xx
        for wt_r, wb_r in ((l0t_r, l0b_r), (l1t_r, l1b_r), (l2t_r, l2b_r)):
            h = _swish(jnp.dot(h, wt_r[...],
                               preferred_element_type=jnp.float32)
                       + wb_r[...]) + h
        h_ref[...] = h

        gids = lax.broadcasted_iota(jnp.int32, (_NB, N_GRAPHS), 1
                                    ).astype(jnp.float32)
        oh = (b_ref[...] == gids).astype(jnp.float32)
        ones = jnp.ones((_NB, HID), jnp.float32)
        dimn = (((0,), (0,)), ((), ()))
        ps0 = lax.dot_general(oh, ones, dimn,
                              preferred_element_type=jnp.float32)
        ps1 = lax.dot_general(oh, h, dimn, preferred_element_type=jnp.float32)
        ps2 = lax.dot_general(oh, h * h, dimn,
                              preferred_element_type=jnp.float32)

        @pl.when(i == 0)
        def _():
            s0_ref[...] = jnp.zeros_like(s0_ref)
            s1_ref[...] = jnp.zeros_like(s1_ref)
            s2_ref[...] = jnp.zeros_like(s2_ref)

        s0_ref[...] += ps0
        s1_ref[...] += ps1
        s2_ref[...] += ps2

    nb = lambda w: pl.BlockSpec((_NB, w), lambda i: (i, 0))
    full = lambda a, b: pl.BlockSpec((a, b), lambda i: (0, 0))
    stat = pl.BlockSpec((N_GRAPHS, HID), lambda i: (0, 0))
    sds = jax.ShapeDtypeStruct
    return pl.pallas_call(
        body,
        grid=grid,
        in_specs=[
            nb(HID), nb(HID), nb(HID), nb(1),
            full(HID, HID), full(1, HID), full(HID, HID),
            full(HID, HID), full(1, HID), full(HID, HID),
            full(HID, HID), full(1, HID), full(HID, HID), full(1, HID),
            full(2 * HID, HID), full(1, HID),
            full(HID, HID), full(1, HID), full(HID, HID), full(1, HID),
            full(HID, HID), full(1, HID),
        ],
        out_specs=[nb(HID), stat, stat, stat],
        out_shape=[
            sds((N_NODES, HID), jnp.float32),
            sds((N_GRAPHS, HID), jnp.float32),
            sds((N_GRAPHS, HID), jnp.float32),
            sds((N_GRAPHS, HID), jnp.float32),
        ],
    )(x, agg1, agg2, batf, c1llt, c1ll_b, c1lrt, c2llt, c2ll_b, c2lrt,
      lin1t, lin1_b, lin2t, lin2_b, lcatt, lcat_b, l0t, l0b, l1t, l1b,
      l2t, l2b)


def _tc_node2(h, batf, s0, s1, s2, norm_w, norm_b, norm_ms, finalt, final_b):
    grid = (N_NODES // _NB,)

    def body(h_ref, b_ref, s0_ref, s1_ref, s2_ref,
             nw_r, nb_r, nms_r, ft_r, fb_r, o_ref):
        cnt = jnp.maximum(s0_ref[...], 1.0)
        mean = s1_ref[...] / cnt
        eh2 = s2_ref[...] / cnt
        ms = nms_r[...]
        var = eh2 - (2.0 * ms - ms * ms) * (mean * mean)
        gids = lax.broadcasted_iota(jnp.int32, (_NB, N_GRAPHS), 1
                                    ).astype(jnp.float32)
        oh = (b_ref[...] == gids).astype(jnp.float32)
        meanb = jnp.dot(oh, mean, preferred_element_type=jnp.float32)
        varb = jnp.dot(oh, var, preferred_element_type=jnp.float32)
        h = h_ref[...]
        centered = h - meanb * ms
        normed = (nw_r[...] * centered * lax.rsqrt(varb + 1e-5) + nb_r[...])
        o_ref[...] = jnp.dot(normed, ft_r[...],
                             preferred_element_type=jnp.float32) + fb_r[...]

    nb = lambda w: pl.BlockSpec((_NB, w), lambda i: (i, 0))
    full = lambda a, b: pl.BlockSpec((a, b), lambda i: (0, 0))
    stat = pl.BlockSpec((N_GRAPHS, HID), lambda i: (0, 0))
    return pl.pallas_call(
        body,
        grid=grid,
        in_specs=[
            nb(HID), nb(1), stat, stat, stat,
            full(1, HID), full(1, HID), full(1, HID),
            full(HID, HID), full(1, HID),
        ],
        out_specs=nb(HID),
        out_shape=jax.ShapeDtypeStruct((N_NODES, HID), jnp.float32),
    )(h, batf, s0, s1, s2, norm_w, norm_b, norm_ms, finalt, final_b)


def kernel(x, feature1, feature2, edge_index, batch,
           f1_w1, f1_w2, f2_w1, f2_w2,
           conv1_ll_w, conv1_ll_b, conv1_lr_w,
           conv2_ll_w, conv2_ll_b, conv2_lr_w,
           lin1_w, lin1_b, lin2_w, lin2_b,
           lin_cat_w, lin_cat_b,
           norm_w, norm_b, norm_ms,
           lins_w, lins_b, final_w, final_b):
    src = edge_index[0].astype(jnp.int32)
    dst = edge_index[1].astype(jnp.int32)
    batf = batch.astype(jnp.float32).reshape(N_NODES, 1)

    g = _sc_gather(x, src)

    cw1t = (f1_w2 @ f1_w1).T
    cw2t = (f2_w2 @ f2_w1).T
    msg1, msg2 = _tc_msg(feature1, feature2, g, cw1t, cw2t)

    agg1, agg2 = _sc_scatter(msg1, msg2, dst)

    r2 = lambda b: b.reshape(1, HID)
    h, s0, s1, s2 = _tc_node1(
        x, agg1, agg2, batf,
        conv1_ll_w.T, r2(conv1_ll_b), conv1_lr_w.T,
        conv2_ll_w.T, r2(conv2_ll_b), conv2_lr_w.T,
        lin1_w.T, r2(lin1_b), lin2_w.T, r2(lin2_b),
        lin_cat_w.T, r2(lin_cat_b),
        lins_w[0].T, lins_b[0].reshape(1, HID),
        lins_w[1].T, lins_b[1].reshape(1, HID),
        lins_w[2].T, lins_b[2].reshape(1, HID),
    )
    out = _tc_node2(h, batf, s0, s1, s2,
                    r2(norm_w), r2(norm_b), r2(norm_ms),
                    final_w.T, final_b.reshape(1, HID))
    return out
